# baseline (speedup 1.0000x reference)
import numpy as np
import jax
import jax.numpy as jnp
from jax.experimental import pallas as pl
from jax.experimental.pallas import tpu as pltpu

_BB = 64


def _roll_rows(a, k):
    if k == 0:
        return a
    return jnp.concatenate([a[k:], a[:k]], axis=0)


def _pool_rows(refs, val):
    w = val.shape[1]
    for j, r in enumerate(refs):
        piece = val[:, 128 * j:min(128 * (j + 1), w)]
        if piece.shape[1] < 128:
            piece = jnp.pad(piece, ((0, 0), (0, 128 - piece.shape[1])))
        r[...] = piece
    return jnp.concatenate(
        [jnp.maximum(r[0::2, :], r[1::2, :]) for r in refs], axis=1)[:, :w]


def _fused_kernel(x_ref, t1e_ref, t1o_ref, b1_ref, t2_ref, b2_ref, w13_ref,
                  fc1b_ref, fc2w_ref, fc2b_ref, o_ref, s2a, s2b, s2c):
    xm = x_ref[...]
    x3 = jnp.concatenate([xm, _roll_rows(xm, 1), _roll_rows(xm, 2)], axis=1)
    oe = jnp.dot(x3, t1e_ref[...], preferred_element_type=jnp.float32)
    oo = jnp.dot(x3, t1o_ref[...], preferred_element_type=jnp.float32)
    p1 = jnp.maximum(
        jnp.maximum(jnp.maximum(oe[:, :300], oe[:, 384:684]),
                    jnp.maximum(oo[:, :300], oo[:, 384:684]))
        + b1_ref[...], 0.0)
    x2 = jnp.concatenate([_roll_rows(p1, k) for k in range(5)], axis=1)
    c2 = jnp.dot(x2, t2_ref[...], preferred_element_type=jnp.float32)
    dm = jnp.maximum(c2[:, :260], c2[:, 384:644])
    p2 = jnp.maximum(_pool_rows((s2a, s2b, s2c), dm) + b2_ref[...], 0.0)
    p2r = p2.reshape(_BB, 16, 260)
    acc = None
    for i2 in range(13):
        d = jnp.dot(p2r[:, i2, :], w13_ref[i2],
                    preferred_element_type=jnp.float32)
        acc = d if acc is None else acc + d
    h1 = jnp.maximum(acc + fc1b_ref[...], 0.0)
    logits = (jnp.dot(h1, fc2w_ref[...], preferred_element_type=jnp.float32)
              + fc2b_ref[...])
    z = logits - jnp.max(logits, axis=-1, keepdims=True)
    o_ref[...] = z - jnp.log(jnp.sum(jnp.exp(z), axis=-1, keepdims=True))


def _build_weights(w1, b1, w2, b2, fc1w):
    e1x = np.zeros((2, 5, 64, 30), np.float32)
    e2x = np.zeros((2, 5, 30, 13), np.float32)
    idx30, idx13 = np.arange(30), np.arange(13)
    for p in range(2):
        for kw in range(5):
            e1x[p, kw, 2 * idx30 + p + kw, idx30] = 1.0
            e2x[p, kw, 2 * idx13 + p + kw, idx13] = 1.0
    w1r = w1.reshape(5, 5, 10)
    w2r = w2.reshape(5, 5, 10, 20)
    base = jnp.pad(jnp.einsum("kqc,pqvw->kvpcw", w1r, e1x
                              ).reshape(5, 64, 2, 300),
                   ((0, 0), (0, 0), (0, 0), (0, 84)))
    t1e = jnp.pad(base, ((0, 1), (0, 0), (0, 0), (0, 0))
                  ).reshape(384, 768)[:, :684]
    t1o = jnp.pad(base, ((1, 0), (0, 0), (0, 0), (0, 0))
                  ).reshape(384, 768)[:, :684]
    t2 = jnp.pad(jnp.einsum("kqic,pqwj->kiwcjp", w2r, e2x).reshape(1500, 260, 2),
                 ((0, 0), (0, 124), (0, 0)))
    t2 = t2.transpose(0, 2, 1).reshape(1500, 768)[:, :644]
    w13 = fc1w.reshape(13, 13, 20, 32).transpose(0, 2, 1, 3).reshape(13, 260, 32)
    b1r = jnp.repeat(b1.reshape(10), 30).reshape(1, 300)
    b2r = jnp.repeat(b2.reshape(20), 13).reshape(1, 260)
    return t1e, t1o, b1r, t2, b2r, w13


def kernel(w1, b1, w2, b2, fc1w, fc1b, fc2w, fc2b, x):
    n = x.shape[0]
    t1e, t1o, b1r, t2, b2r, w13 = _build_weights(w1, b1, w2, b2, fc1w)
    xm = x.reshape(n * 32, 128)
    flops = n * 2 * (60 * 60 * 10 * 25 + 26 * 26 * 20 * 250
                     + 3380 * 32 + 32 * 6)
    bytes_accessed = 4 * (xm.size + t1e.size * 2 + t2.size + w13.size + n * 6)
    out = pl.pallas_call(
        _fused_kernel,
        out_shape=jax.ShapeDtypeStruct((n, 6), jnp.float32),
        grid=(n // _BB,),
        in_specs=[
            pl.BlockSpec((_BB * 32, 128), lambda i: (i, 0)),
            pl.BlockSpec((384, 684), lambda i: (0, 0)),
            pl.BlockSpec((384, 684), lambda i: (0, 0)),
            pl.BlockSpec((1, 300), lambda i: (0, 0)),
            pl.BlockSpec((1500, 644), lambda i: (0, 0)),
            pl.BlockSpec((1, 260), lambda i: (0, 0)),
            pl.BlockSpec((13, 260, 32), lambda i: (0, 0, 0)),
            pl.BlockSpec((1, 32), lambda i: (0, 0)),
            pl.BlockSpec((32, 6), lambda i: (0, 0)),
            pl.BlockSpec((1, 6), lambda i: (0, 0)),
        ],
        out_specs=pl.BlockSpec((_BB, 6), lambda i: (i, 0)),
        scratch_shapes=[
            pltpu.VMEM((_BB * 32, 128), jnp.float32),
            pltpu.VMEM((_BB * 32, 128), jnp.float32),
            pltpu.VMEM((_BB * 32, 128), jnp.float32),
        ],
        compiler_params=pltpu.CompilerParams(
            dimension_semantics=("parallel",),
            vmem_limit_bytes=60 * 1024 * 1024,
        ),
        cost_estimate=pl.CostEstimate(
            flops=flops, transcendentals=7 * n, bytes_accessed=bytes_accessed),
    )(xm, t1e, t1o, b1r, t2, b2r, w13, fc1b, fc2w, fc2b)
    return out


# speedup vs baseline: 1.0780x; 1.0780x over previous
import numpy as np
import jax
import jax.numpy as jnp
from jax.experimental import pallas as pl
from jax.experimental.pallas import tpu as pltpu

_BB = 64


def _roll_rows(a, k):
    if k == 0:
        return a
    return jnp.concatenate([a[k:], a[:k]], axis=0)


def _pool_rows(refs, val):
    w = val.shape[1]
    for j, r in enumerate(refs):
        piece = val[:, 128 * j:min(128 * (j + 1), w)]
        if piece.shape[1] < 128:
            piece = jnp.pad(piece, ((0, 0), (0, 128 - piece.shape[1])))
        r[...] = piece
    return jnp.concatenate(
        [jnp.maximum(r[0::2, :], r[1::2, :]) for r in refs], axis=1)[:, :w]


def _chain(xm, t1_ref, b1_ref, t2_ref, b2_ref, w13_ref,
           fc1b_ref, fc2w_ref, fc2b_ref, s1, s2, nb):
    x5 = jnp.concatenate([_roll_rows(xm, k) for k in range(5)], axis=1)
    c1 = jnp.dot(x5, t1_ref[...], preferred_element_type=jnp.float32)
    wm = jnp.maximum(c1[:, :300], c1[:, 384:684])
    p1 = jnp.maximum(_pool_rows(s1, wm) + b1_ref[...], 0.0)
    x2 = jnp.concatenate([_roll_rows(p1, k) for k in range(5)], axis=1)
    c2 = jnp.dot(x2, t2_ref[...], preferred_element_type=jnp.float32)
    dm = jnp.maximum(c2[:, :260], c2[:, 384:644])
    p2 = jnp.maximum(_pool_rows(s2, dm) + b2_ref[...], 0.0)
    p2r = p2.reshape(nb, 16, 260)
    acc = None
    for i2 in range(13):
        d = jnp.dot(p2r[:, i2, :], w13_ref[i2],
                    preferred_element_type=jnp.float32)
        acc = d if acc is None else acc + d
    h1 = jnp.maximum(acc + fc1b_ref[...], 0.0)
    logits = (jnp.dot(h1, fc2w_ref[...], preferred_element_type=jnp.float32)
              + fc2b_ref[...])
    z = logits - jnp.max(logits, axis=-1, keepdims=True)
    return z - jnp.log(jnp.sum(jnp.exp(z), axis=-1, keepdims=True))


def _fused_kernel(x_ref, t1_ref, b1_ref, t2_ref, b2_ref, w13_ref,
                  fc1b_ref, fc2w_ref, fc2b_ref, o_ref,
                  s1a, s1b, s1c, s2a, s2b, s2c,
                  s1d, s1e, s1f, s2d, s2e, s2f):
    hb = _BB // 2
    args = (t1_ref, b1_ref, t2_ref, b2_ref, w13_ref,
            fc1b_ref, fc2w_ref, fc2b_ref)
    o_ref[:hb, :] = _chain(x_ref[:hb * 64, :], *args,
                           (s1a, s1b, s1c), (s2a, s2b, s2c), hb)
    o_ref[hb:, :] = _chain(x_ref[hb * 64:, :], *args,
                           (s1d, s1e, s1f), (s2d, s2e, s2f), hb)


def _build_weights(w1, b1, w2, b2, fc1w):
    e1x = np.zeros((2, 5, 64, 30), np.float32)
    e2x = np.zeros((2, 5, 30, 13), np.float32)
    idx30, idx13 = np.arange(30), np.arange(13)
    for p in range(2):
        for kw in range(5):
            e1x[p, kw, 2 * idx30 + p + kw, idx30] = 1.0
            e2x[p, kw, 2 * idx13 + p + kw, idx13] = 1.0
    w1r = w1.reshape(5, 5, 10)
    w2r = w2.reshape(5, 5, 10, 20)
    t1 = jnp.pad(jnp.einsum("kqc,pqvw->kvcwp", w1r, e1x).reshape(320, 300, 2),
                 ((0, 0), (0, 84), (0, 0)))
    t1 = t1.transpose(0, 2, 1).reshape(320, 768)[:, :684]
    t2 = jnp.pad(jnp.einsum("kqic,pqwj->kiwcjp", w2r, e2x).reshape(1500, 260, 2),
                 ((0, 0), (0, 124), (0, 0)))
    t2 = t2.transpose(0, 2, 1).reshape(1500, 768)[:, :644]
    w13 = fc1w.reshape(13, 13, 20, 32).transpose(0, 2, 1, 3).reshape(13, 260, 32)
    b1r = jnp.repeat(b1.reshape(10), 30).reshape(1, 300)
    b2r = jnp.repeat(b2.reshape(20), 13).reshape(1, 260)
    return t1, b1r, t2, b2r, w13


def kernel(w1, b1, w2, b2, fc1w, fc1b, fc2w, fc2b, x):
    n = x.shape[0]
    t1, b1r, t2, b2r, w13 = _build_weights(w1, b1, w2, b2, fc1w)
    xm = x.reshape(n * 64, 64)
    flops = n * 2 * (60 * 60 * 10 * 25 + 26 * 26 * 20 * 250
                     + 3380 * 32 + 32 * 6)
    bytes_accessed = 4 * (xm.size + t1.size + t2.size + w13.size + n * 6)
    out = pl.pallas_call(
        _fused_kernel,
        out_shape=jax.ShapeDtypeStruct((n, 6), jnp.float32),
        grid=(n // _BB,),
        in_specs=[
            pl.BlockSpec((_BB * 64, 64), lambda i: (i, 0)),
            pl.BlockSpec((320, 684), lambda i: (0, 0)),
            pl.BlockSpec((1, 300), lambda i: (0, 0)),
            pl.BlockSpec((1500, 644), lambda i: (0, 0)),
            pl.BlockSpec((1, 260), lambda i: (0, 0)),
            pl.BlockSpec((13, 260, 32), lambda i: (0, 0, 0)),
            pl.BlockSpec((1, 32), lambda i: (0, 0)),
            pl.BlockSpec((32, 6), lambda i: (0, 0)),
            pl.BlockSpec((1, 6), lambda i: (0, 0)),
        ],
        out_specs=pl.BlockSpec((_BB, 6), lambda i: (i, 0)),
        scratch_shapes=(
            [pltpu.VMEM((_BB * 32, 128), jnp.float32)] * 3
            + [pltpu.VMEM((_BB * 16, 128), jnp.float32)] * 3) * 2,
        compiler_params=pltpu.CompilerParams(
            dimension_semantics=("parallel",),
            vmem_limit_bytes=60 * 1024 * 1024,
        ),
        cost_estimate=pl.CostEstimate(
            flops=flops, transcendentals=7 * n, bytes_accessed=bytes_accessed),
    )(xm, t1, b1r, t2, b2r, w13, fc1b, fc2w, fc2b)
    return out


# speedup vs baseline: 1.0878x; 1.0090x over previous
import numpy as np
import jax
import jax.numpy as jnp
from jax.experimental import pallas as pl
from jax.experimental.pallas import tpu as pltpu

_BB = 64


def _roll_rows(a, k):
    if k == 0:
        return a
    return jnp.concatenate([a[k:], a[:k]], axis=0)


def _pool_rows(refs, val):
    w = val.shape[1]
    for j, r in enumerate(refs):
        piece = val[:, 128 * j:min(128 * (j + 1), w)]
        if piece.shape[1] < 128:
            piece = jnp.pad(piece, ((0, 0), (0, 128 - piece.shape[1])))
        r[...] = piece
    return jnp.concatenate(
        [jnp.maximum(r[0::2, :], r[1::2, :]) for r in refs], axis=1)[:, :w]


def _fused_kernel(x_ref, t1_ref, b1_ref, t2_ref, b2_ref, w13_ref,
                  fc1b_ref, fc2w_ref, fc2b_ref, o_ref,
                  s1a, s1b, s1c, s2a, s2b, s2c):
    xm = x_ref[...]
    x5 = jnp.concatenate([_roll_rows(xm, k) for k in range(5)], axis=1)
    c1 = jnp.dot(x5, t1_ref[...], preferred_element_type=jnp.float32)
    wm = jnp.maximum(c1[:, :300], c1[:, 384:684])
    p1 = jnp.maximum(_pool_rows((s1a, s1b, s1c), wm) + b1_ref[...], 0.0)
    x2 = jnp.concatenate([_roll_rows(p1, k) for k in range(5)], axis=1)
    c2 = jnp.dot(x2, t2_ref[...], preferred_element_type=jnp.float32)
    dm = jnp.maximum(c2[:, :260], c2[:, 384:644])
    p2 = jnp.maximum(_pool_rows((s2a, s2b, s2c), dm) + b2_ref[...], 0.0)
    p2r = p2.reshape(_BB, 16, 260)
    acc = None
    for i2 in range(13):
        d = jnp.dot(p2r[:, i2, :], w13_ref[i2],
                    preferred_element_type=jnp.float32)
        acc = d if acc is None else acc + d
    h1 = jnp.maximum(acc + fc1b_ref[...], 0.0)
    logits = (jnp.dot(h1, fc2w_ref[...], preferred_element_type=jnp.float32)
              + fc2b_ref[...])
    z = logits - jnp.max(logits, axis=-1, keepdims=True)
    o_ref[...] = z - jnp.log(jnp.sum(jnp.exp(z), axis=-1, keepdims=True))


def _build_weights(w1, b1, w2, b2, fc1w):
    e1x = np.zeros((2, 5, 64, 30), np.float32)
    e2x = np.zeros((2, 5, 30, 13), np.float32)
    idx30, idx13 = np.arange(30), np.arange(13)
    for p in range(2):
        for kw in range(5):
            e1x[p, kw, 2 * idx30 + p + kw, idx30] = 1.0
            e2x[p, kw, 2 * idx13 + p + kw, idx13] = 1.0
    w1r = w1.reshape(5, 5, 10)
    w2r = w2.reshape(5, 5, 10, 20)
    t1 = jnp.pad(jnp.einsum("kqc,pqvw->kvpcw", w1r, e1x).reshape(320, 2, 300),
                 ((0, 0), (0, 0), (0, 84))).reshape(320, 768)
    t2 = jnp.pad(jnp.einsum("kqic,pqwj->kiwpcj", w2r, e2x).reshape(1500, 2, 260),
                 ((0, 0), (0, 0), (0, 124))).reshape(1500, 768)
    w13 = fc1w.reshape(13, 13, 20, 32).transpose(0, 2, 1, 3).reshape(13, 260, 32)
    b1r = jnp.repeat(b1.reshape(10), 30).reshape(1, 300)
    b2r = jnp.repeat(b2.reshape(20), 13).reshape(1, 260)
    return t1, b1r, t2, b2r, w13


def kernel(w1, b1, w2, b2, fc1w, fc1b, fc2w, fc2b, x):
    n = x.shape[0]
    t1, b1r, t2, b2r, w13 = _build_weights(w1, b1, w2, b2, fc1w)
    xm = x.reshape(n * 64, 64)
    flops = n * 2 * (60 * 60 * 10 * 25 + 26 * 26 * 20 * 250
                     + 3380 * 32 + 32 * 6)
    bytes_accessed = 4 * (xm.size + t1.size + t2.size + w13.size + n * 6)
    out = pl.pallas_call(
        _fused_kernel,
        out_shape=jax.ShapeDtypeStruct((n, 6), jnp.float32),
        grid=(n // _BB,),
        in_specs=[
            pl.BlockSpec((_BB * 64, 64), lambda i: (i, 0)),
            pl.BlockSpec((320, 768), lambda i: (0, 0)),
            pl.BlockSpec((1, 300), lambda i: (0, 0)),
            pl.BlockSpec((1500, 768), lambda i: (0, 0)),
            pl.BlockSpec((1, 260), lambda i: (0, 0)),
            pl.BlockSpec((13, 260, 32), lambda i: (0, 0, 0)),
            pl.BlockSpec((1, 32), lambda i: (0, 0)),
            pl.BlockSpec((32, 6), lambda i: (0, 0)),
            pl.BlockSpec((1, 6), lambda i: (0, 0)),
        ],
        out_specs=pl.BlockSpec((_BB, 6), lambda i: (i, 0)),
        scratch_shapes=[
            pltpu.VMEM((_BB * 64, 128), jnp.float32),
            pltpu.VMEM((_BB * 64, 128), jnp.float32),
            pltpu.VMEM((_BB * 64, 128), jnp.float32),
            pltpu.VMEM((_BB * 32, 128), jnp.float32),
            pltpu.VMEM((_BB * 32, 128), jnp.float32),
            pltpu.VMEM((_BB * 32, 128), jnp.float32),
        ],
        compiler_params=pltpu.CompilerParams(
            dimension_semantics=("parallel",),
            vmem_limit_bytes=60 * 1024 * 1024,
        ),
        cost_estimate=pl.CostEstimate(
            flops=flops, transcendentals=7 * n, bytes_accessed=bytes_accessed),
    )(xm, t1, b1r, t2, b2r, w13, fc1b, fc2w, fc2b)
    return out


# speedup vs baseline: 1.0889x; 1.0010x over previous
import numpy as np
import jax
import jax.numpy as jnp
from jax.experimental import pallas as pl
from jax.experimental.pallas import tpu as pltpu

_BB = 64


def _roll_rows(a, k):
    if k == 0:
        return a
    return jnp.concatenate([a[k:], a[:k]], axis=0)


def _pool_rows(refs, val):
    w = val.shape[1]
    for j, r in enumerate(refs):
        piece = val[:, 128 * j:min(128 * (j + 1), w)]
        if piece.shape[1] < 128:
            piece = jnp.pad(piece, ((0, 0), (0, 128 - piece.shape[1])))
        r[...] = piece
    return jnp.concatenate(
        [jnp.maximum(r[0::2, :], r[1::2, :]) for r in refs], axis=1)[:, :w]


def _fused_kernel(x_ref, t1_ref, b1_ref, t2_ref, b2_ref, w13_ref,
                  fc1b_ref, fc2w_ref, fc2b_ref, o_ref,
                  s1a, s1b, s1c, s2a, s2b, s2c):
    xm = x_ref[...]
    x5 = jnp.concatenate([_roll_rows(xm, k) for k in range(5)], axis=1)
    c1 = jnp.dot(x5, t1_ref[...], preferred_element_type=jnp.float32)
    wm = jnp.maximum(c1[:, :300], c1[:, 384:684])
    p1 = jnp.maximum(_pool_rows((s1a, s1b, s1c), wm) + b1_ref[...], 0.0)
    x2 = jnp.concatenate([_roll_rows(p1, k) for k in range(5)], axis=1)
    c2 = jnp.dot(x2, t2_ref[...], preferred_element_type=jnp.float32)
    dm = jnp.maximum(c2[:, :260], c2[:, 260:520])
    p2 = jnp.maximum(_pool_rows((s2a, s2b, s2c), dm) + b2_ref[...], 0.0)
    p2r = p2.reshape(_BB, 16, 260)
    acc = None
    for i2 in range(13):
        d = jnp.dot(p2r[:, i2, :], w13_ref[i2],
                    preferred_element_type=jnp.float32)
        acc = d if acc is None else acc + d
    h1 = jnp.maximum(acc + fc1b_ref[...], 0.0)
    logits = (jnp.dot(h1, fc2w_ref[...], preferred_element_type=jnp.float32)
              + fc2b_ref[...])
    z = logits - jnp.max(logits, axis=-1, keepdims=True)
    o_ref[...] = z - jnp.log(jnp.sum(jnp.exp(z), axis=-1, keepdims=True))


def _build_weights(w1, b1, w2, b2, fc1w):
    e1x = np.zeros((2, 5, 64, 30), np.float32)
    e2x = np.zeros((2, 5, 30, 13), np.float32)
    idx30, idx13 = np.arange(30), np.arange(13)
    for p in range(2):
        for kw in range(5):
            e1x[p, kw, 2 * idx30 + p + kw, idx30] = 1.0
            e2x[p, kw, 2 * idx13 + p + kw, idx13] = 1.0
    w1r = w1.reshape(5, 5, 10)
    w2r = w2.reshape(5, 5, 10, 20)
    t1 = jnp.pad(jnp.einsum("kqc,pqvw->kvcwp", w1r, e1x).reshape(320, 300, 2),
                 ((0, 0), (0, 84), (0, 0)))
    t1 = t1.transpose(0, 2, 1).reshape(320, 768)[:, :684]
    t2 = jnp.einsum("kqic,pqwj->kiwpcj", w2r, e2x).reshape(1500, 520)
    w13 = fc1w.reshape(13, 13, 20, 32).transpose(0, 2, 1, 3).reshape(13, 260, 32)
    b1r = jnp.repeat(b1.reshape(10), 30).reshape(1, 300)
    b2r = jnp.repeat(b2.reshape(20), 13).reshape(1, 260)
    return t1, b1r, t2, b2r, w13


def kernel(w1, b1, w2, b2, fc1w, fc1b, fc2w, fc2b, x):
    n = x.shape[0]
    t1, b1r, t2, b2r, w13 = _build_weights(w1, b1, w2, b2, fc1w)
    xm = x.reshape(n * 64, 64)
    flops = n * 2 * (60 * 60 * 10 * 25 + 26 * 26 * 20 * 250
                     + 3380 * 32 + 32 * 6)
    bytes_accessed = 4 * (xm.size + t1.size + t2.size + w13.size + n * 6)
    out = pl.pallas_call(
        _fused_kernel,
        out_shape=jax.ShapeDtypeStruct((n, 6), jnp.float32),
        grid=(n // _BB,),
        in_specs=[
            pl.BlockSpec((_BB * 64, 64), lambda i: (i, 0)),
            pl.BlockSpec((320, 684), lambda i: (0, 0)),
            pl.BlockSpec((1, 300), lambda i: (0, 0)),
            pl.BlockSpec((1500, 520), lambda i: (0, 0)),
            pl.BlockSpec((1, 260), lambda i: (0, 0)),
            pl.BlockSpec((13, 260, 32), lambda i: (0, 0, 0)),
            pl.BlockSpec((1, 32), lambda i: (0, 0)),
            pl.BlockSpec((32, 6), lambda i: (0, 0)),
            pl.BlockSpec((1, 6), lambda i: (0, 0)),
        ],
        out_specs=pl.BlockSpec((_BB, 6), lambda i: (i, 0)),
        scratch_shapes=[
            pltpu.VMEM((_BB * 64, 128), jnp.float32),
            pltpu.VMEM((_BB * 64, 128), jnp.float32),
            pltpu.VMEM((_BB * 64, 128), jnp.float32),
            pltpu.VMEM((_BB * 32, 128), jnp.float32),
            pltpu.VMEM((_BB * 32, 128), jnp.float32),
            pltpu.VMEM((_BB * 32, 128), jnp.float32),
        ],
        compiler_params=pltpu.CompilerParams(
            dimension_semantics=("parallel",),
            vmem_limit_bytes=60 * 1024 * 1024,
        ),
        cost_estimate=pl.CostEstimate(
            flops=flops, transcendentals=7 * n, bytes_accessed=bytes_accessed),
    )(xm, t1, b1r, t2, b2r, w13, fc1b, fc2w, fc2b)
    return out


# speedup vs baseline: 1.1031x; 1.0130x over previous
import numpy as np
import jax
import jax.numpy as jnp
from jax.experimental import pallas as pl
from jax.experimental.pallas import tpu as pltpu

_BB = 64


def _roll_rows(a, k):
    if k == 0:
        return a
    return jnp.concatenate([a[k:], a[:k]], axis=0)


def _pool_rows(refs, val):
    w = val.shape[1]
    for j, r in enumerate(refs):
        piece = val[:, 128 * j:min(128 * (j + 1), w)]
        if piece.shape[1] < 128:
            piece = jnp.pad(piece, ((0, 0), (0, 128 - piece.shape[1])))
        r[...] = piece
    return jnp.concatenate(
        [jnp.maximum(r[0::2, :], r[1::2, :]) for r in refs], axis=1)[:, :w]


def _fused_kernel(x_ref, t1_ref, b1_ref, t2_ref, b2_ref, w13_ref,
                  fc1b_ref, fc2w_ref, fc2b_ref, o_ref,
                  s1a, s1b, s1c, s2a, s2b, s2c):
    xm = x_ref[...]
    x5 = jnp.concatenate([_roll_rows(xm, k) for k in range(5)], axis=1)
    c1 = jnp.dot(x5, t1_ref[...], preferred_element_type=jnp.float32)
    wm = jnp.maximum(c1[:, :300], c1[:, 384:684])
    wm2 = jnp.maximum(jnp.maximum(wm, _roll_rows(wm, 1)) + b1_ref[...], 0.0)
    nr = wm2.shape[0]
    for j, r in enumerate((s1a, s1b, s1c)):
        piece = wm2[:, 128 * j:min(128 * (j + 1), 300)]
        if piece.shape[1] < 128:
            piece = jnp.pad(piece, ((0, 0), (0, 128 - piece.shape[1])))
        r[:nr, :] = piece
    x2 = jnp.concatenate(
        [jnp.concatenate([s1a[2 * k:2 * k + nr:2, :],
                          s1b[2 * k:2 * k + nr:2, :],
                          s1c[2 * k:2 * k + nr:2, :]], axis=1)[:, :300]
         for k in range(5)], axis=1)
    c2 = jnp.dot(x2, t2_ref[...], preferred_element_type=jnp.float32)
    dm = jnp.maximum(c2[:, :260], c2[:, 384:644])
    p2 = jnp.maximum(_pool_rows((s2a, s2b, s2c), dm) + b2_ref[...], 0.0)
    p2r = p2.reshape(_BB, 16, 260)
    acc = None
    for i2 in range(13):
        d = jnp.dot(p2r[:, i2, :], w13_ref[i2],
                    preferred_element_type=jnp.float32)
        acc = d if acc is None else acc + d
    h1 = jnp.maximum(acc + fc1b_ref[...], 0.0)
    logits = (jnp.dot(h1, fc2w_ref[...], preferred_element_type=jnp.float32)
              + fc2b_ref[...])
    z = logits - jnp.max(logits, axis=-1, keepdims=True)
    o_ref[...] = z - jnp.log(jnp.sum(jnp.exp(z), axis=-1, keepdims=True))


def _build_weights(w1, b1, w2, b2, fc1w):
    e1x = np.zeros((2, 5, 64, 30), np.float32)
    e2x = np.zeros((2, 5, 30, 13), np.float32)
    idx30, idx13 = np.arange(30), np.arange(13)
    for p in range(2):
        for kw in range(5):
            e1x[p, kw, 2 * idx30 + p + kw, idx30] = 1.0
            e2x[p, kw, 2 * idx13 + p + kw, idx13] = 1.0
    w1r = w1.reshape(5, 5, 10)
    w2r = w2.reshape(5, 5, 10, 20)
    t1 = jnp.pad(jnp.einsum("kqc,pqvw->kvcwp", w1r, e1x).reshape(320, 300, 2),
                 ((0, 0), (0, 84), (0, 0)))
    t1 = t1.transpose(0, 2, 1).reshape(320, 768)[:, :684]
    t2 = jnp.pad(jnp.einsum("kqic,pqwj->kiwcjp", w2r, e2x).reshape(1500, 260, 2),
                 ((0, 0), (0, 124), (0, 0)))
    t2 = t2.transpose(0, 2, 1).reshape(1500, 768)[:, :644]
    w13 = fc1w.reshape(13, 13, 20, 32).transpose(0, 2, 1, 3).reshape(13, 260, 32)
    b1r = jnp.repeat(b1.reshape(10), 30).reshape(1, 300)
    b2r = jnp.repeat(b2.reshape(20), 13).reshape(1, 260)
    return t1, b1r, t2, b2r, w13


def kernel(w1, b1, w2, b2, fc1w, fc1b, fc2w, fc2b, x):
    n = x.shape[0]
    t1, b1r, t2, b2r, w13 = _build_weights(w1, b1, w2, b2, fc1w)
    xm = x.reshape(n * 64, 64)
    flops = n * 2 * (60 * 60 * 10 * 25 + 26 * 26 * 20 * 250
                     + 3380 * 32 + 32 * 6)
    bytes_accessed = 4 * (xm.size + t1.size + t2.size + w13.size + n * 6)
    out = pl.pallas_call(
        _fused_kernel,
        out_shape=jax.ShapeDtypeStruct((n, 6), jnp.float32),
        grid=(n // _BB,),
        in_specs=[
            pl.BlockSpec((_BB * 64, 64), lambda i: (i, 0)),
            pl.BlockSpec((320, 684), lambda i: (0, 0)),
            pl.BlockSpec((1, 300), lambda i: (0, 0)),
            pl.BlockSpec((1500, 644), lambda i: (0, 0)),
            pl.BlockSpec((1, 260), lambda i: (0, 0)),
            pl.BlockSpec((13, 260, 32), lambda i: (0, 0, 0)),
            pl.BlockSpec((1, 32), lambda i: (0, 0)),
            pl.BlockSpec((32, 6), lambda i: (0, 0)),
            pl.BlockSpec((1, 6), lambda i: (0, 0)),
        ],
        out_specs=pl.BlockSpec((_BB, 6), lambda i: (i, 0)),
        scratch_shapes=[
            pltpu.VMEM((_BB * 64 + 16, 128), jnp.float32),
            pltpu.VMEM((_BB * 64 + 16, 128), jnp.float32),
            pltpu.VMEM((_BB * 64 + 16, 128), jnp.float32),
            pltpu.VMEM((_BB * 32, 128), jnp.float32),
            pltpu.VMEM((_BB * 32, 128), jnp.float32),
            pltpu.VMEM((_BB * 32, 128), jnp.float32),
        ],
        compiler_params=pltpu.CompilerParams(
            dimension_semantics=("parallel",),
            vmem_limit_bytes=60 * 1024 * 1024,
        ),
        cost_estimate=pl.CostEstimate(
            flops=flops, transcendentals=7 * n, bytes_accessed=bytes_accessed),
    )(xm, t1, b1r, t2, b2r, w13, fc1b, fc2w, fc2b)
    return out
